# baseline (speedup 1.0000x reference)
import functools

import jax
import jax.numpy as jnp
from jax.experimental import pallas as pl
from jax.experimental.pallas import tpu as pltpu

D_MODEL_C = 2048
N_TIERS_C = 8
B_C = 8192
LB_COEFF_C = 0.01
BLK = 128


def _router_block(tier_ref, q_ref, merged_ref, rw_ref, lb_ref, acc_ref):
    step = pl.program_id(0)
    nsteps = pl.num_programs(0)

    q = q_ref[...]
    tiers = tier_ref[...]

    tiers_r = tiers.astype(jnp.bfloat16).astype(jnp.float32)
    q_r = q.astype(jnp.bfloat16).astype(jnp.float32)
    scores = jnp.sum(tiers_r * q_r[None, :, :], axis=2)

    tier_iota = jax.lax.broadcasted_iota(jnp.int32, scores.shape, 0)

    v0 = jnp.max(scores, axis=0)
    i0 = jnp.argmax(scores, axis=0)
    masked = jnp.where(tier_iota == i0[None, :], -jnp.inf, scores)
    v1 = jnp.max(masked, axis=0)
    i1 = jnp.argmax(masked, axis=0)

    w1 = jax.nn.sigmoid(v1 - v0)
    w0 = 1.0 - w1

    rw = jnp.where(tier_iota == i0[None, :], w0[None, :], 0.0) + jnp.where(
        tier_iota == i1[None, :], w1[None, :], 0.0
    )
    rw_ref[:, pl.ds(step * BLK, BLK)] = rw

    merged_ref[pl.ds((step % 2) * BLK, BLK), :] = jnp.sum(
        rw[:, :, None] * tiers, axis=0
    )

    @pl.when(step == 0)
    def _init():
        acc_ref[...] = jnp.zeros_like(acc_ref)

    acc_ref[...] += rw

    @pl.when(step == nsteps - 1)
    def _finish():
        frac = jnp.sum(acc_ref[...], axis=1) * (1.0 / B_C)
        mean = jnp.mean(frac)
        dev = frac - mean
        var = jnp.sum(dev * dev) * (1.0 / (N_TIERS_C - 1))
        lb_ref[...] = jnp.reshape(LB_COEFF_C * var, (1, 1))


@functools.partial(jax.jit, static_argnames=())
def _router(tier_outputs, query):
    nblocks = B_C // BLK
    merged, rw_t, lb = pl.pallas_call(
        _router_block,
        grid=(nblocks,),
        in_specs=[
            pl.BlockSpec((N_TIERS_C, BLK, D_MODEL_C), lambda i: (0, i, 0)),
            pl.BlockSpec((BLK, D_MODEL_C), lambda i: (i, 0)),
        ],
        out_specs=[
            pl.BlockSpec((2 * BLK, D_MODEL_C), lambda i: (i // 2, 0)),
            pl.BlockSpec((N_TIERS_C, B_C), lambda i: (0, 0)),
            pl.BlockSpec((1, 1), lambda i: (0, 0)),
        ],
        out_shape=[
            jax.ShapeDtypeStruct((B_C, D_MODEL_C), jnp.float32),
            jax.ShapeDtypeStruct((N_TIERS_C, B_C), jnp.float32),
            jax.ShapeDtypeStruct((1, 1), jnp.float32),
        ],
        scratch_shapes=[pltpu.VMEM((N_TIERS_C, BLK), jnp.float32)],
        compiler_params=pltpu.CompilerParams(
            dimension_semantics=("arbitrary",),
        ),
    )(tier_outputs, query)
    return merged, rw_t.T, lb[0, 0]


def kernel(tier_outputs, query):
    tier_outputs = tier_outputs.astype(jnp.float32)
    query = query.astype(jnp.float32)
    return _router(tier_outputs, query)


# speedup vs baseline: 1.0051x; 1.0051x over previous
import functools

import jax
import jax.numpy as jnp
from jax.experimental import pallas as pl
from jax.experimental.pallas import tpu as pltpu

D_MODEL_C = 2048
N_TIERS_C = 8
B_C = 8192
LB_COEFF_C = 0.01
BLK = 256


def _router_block(tier_ref, q_ref, merged_ref, rw_ref, lb_ref, acc_ref):
    step = pl.program_id(0)
    nsteps = pl.num_programs(0)

    q = q_ref[...]
    tiers = tier_ref[...]

    tiers_r = tiers.astype(jnp.bfloat16).astype(jnp.float32)
    q_r = q.astype(jnp.bfloat16).astype(jnp.float32)
    scores = jnp.sum(tiers_r * q_r[None, :, :], axis=2)

    tier_iota = jax.lax.broadcasted_iota(jnp.int32, scores.shape, 0)

    v0 = jnp.max(scores, axis=0)
    i0 = jnp.argmax(scores, axis=0)
    masked = jnp.where(tier_iota == i0[None, :], -jnp.inf, scores)
    v1 = jnp.max(masked, axis=0)
    i1 = jnp.argmax(masked, axis=0)

    w1 = jax.nn.sigmoid(v1 - v0)
    w0 = 1.0 - w1

    rw = jnp.where(tier_iota == i0[None, :], w0[None, :], 0.0) + jnp.where(
        tier_iota == i1[None, :], w1[None, :], 0.0
    )
    rw_ref[:, pl.ds(step * BLK, BLK)] = rw

    merged_ref[pl.ds((step % 2) * BLK, BLK), :] = jnp.sum(
        rw[:, :, None] * tiers, axis=0
    )

    @pl.when(step == 0)
    def _init():
        acc_ref[...] = jnp.zeros_like(acc_ref)

    acc_ref[...] += rw

    @pl.when(step == nsteps - 1)
    def _finish():
        frac = jnp.sum(acc_ref[...], axis=1) * (1.0 / B_C)
        mean = jnp.mean(frac)
        dev = frac - mean
        var = jnp.sum(dev * dev) * (1.0 / (N_TIERS_C - 1))
        lb_ref[...] = jnp.reshape(LB_COEFF_C * var, (1, 1))


@functools.partial(jax.jit, static_argnames=())
def _router(tier_outputs, query):
    nblocks = B_C // BLK
    merged, rw_t, lb = pl.pallas_call(
        _router_block,
        grid=(nblocks,),
        in_specs=[
            pl.BlockSpec((N_TIERS_C, BLK, D_MODEL_C), lambda i: (0, i, 0)),
            pl.BlockSpec((BLK, D_MODEL_C), lambda i: (i, 0)),
        ],
        out_specs=[
            pl.BlockSpec((2 * BLK, D_MODEL_C), lambda i: (i // 2, 0)),
            pl.BlockSpec((N_TIERS_C, B_C), lambda i: (0, 0)),
            pl.BlockSpec((1, 1), lambda i: (0, 0)),
        ],
        out_shape=[
            jax.ShapeDtypeStruct((B_C, D_MODEL_C), jnp.float32),
            jax.ShapeDtypeStruct((N_TIERS_C, B_C), jnp.float32),
            jax.ShapeDtypeStruct((1, 1), jnp.float32),
        ],
        scratch_shapes=[pltpu.VMEM((N_TIERS_C, BLK), jnp.float32)],
        compiler_params=pltpu.CompilerParams(
            dimension_semantics=("arbitrary",),
        ),
    )(tier_outputs, query)
    return merged, rw_t.T, lb[0, 0]


def kernel(tier_outputs, query):
    tier_outputs = tier_outputs.astype(jnp.float32)
    query = query.astype(jnp.float32)
    return _router(tier_outputs, query)
